# baseline (speedup 1.0000x reference)
import math

import jax
import jax.numpy as jnp
import numpy as np
from jax import lax
from jax.experimental import pallas as pl
from jax.experimental.pallas import tpu as pltpu
from jax.experimental.pallas import tpu_sc as plsc

N = 10000
E = 160000
E2 = 2 * E
CH = 256
HC = CH // 2
G = 64
ALPHA = 0.1
BETA = math.log(0.5 / 8 + 1.0)
TBL = N * N

NC = 2
NS = 16
NW = NC * NS
WE = 128
ROWSP = 2560
E2P = ROWSP * WE
RPW = ROWSP // NW
RPT = ROWSP // NS
GA = 8
NGA = RPW // GA
GD = 8
NGD = RPT // GD
NP = 10016
DSP = 12000

NB = 10
_Z = np.int32(0)
BN = N // NB

_mesh = plsc.VectorSubcoreMesh(core_axis_name="c", subcore_axis_name="s")

f32 = jnp.float32
i32 = jnp.int32


def c32(v):
    return np.int32(v)


def _iota16():
    return lax.iota(i32, 16)


def _dedup_scatter_body(keys_hbm, table_hbm, kbuf, ibuf, sem):
    c = lax.axis_index("c")
    s = lax.axis_index("s")
    wid = s * c32(NC) + c
    row0w = pl.multiple_of(wid * c32(RPW), 8)

    pltpu.sync_copy(keys_hbm.at[pl.ds(row0w, RPW)], kbuf)

    def fill_row(j, state):
        rr, off = state
        for q in range(WE // 16):
            ibuf[rr, pl.ds(q * 16, 16)] = off + c32(q * 16) + _iota16()
        return (rr + c32(1), off + c32(WE))
    lax.fori_loop(np.int32(0), np.int32(RPW), fill_row,
                  (np.int32(0), row0w * c32(WE)))

    cps = [
        pltpu.async_copy(ibuf.at[c32(r)], table_hbm.at[kbuf.at[c32(r)]], sem)
        for r in range(RPW)
    ]
    for cp in cps:
        cp.wait()


def _dedup_scatter(keys2d):
    return pl.kernel(
        _dedup_scatter_body,
        out_type=jax.ShapeDtypeStruct((TBL + 16,), i32),
        mesh=_mesh,
        scratch_types=[
            pltpu.VMEM((RPW, WE), i32),
            pltpu.VMEM((RPW, WE), i32),
            pltpu.SemaphoreType.DMA,
        ],
    )(keys2d)


def _dedup_gather_body(keys_hbm, table_hbm, dst_hbm, dst_eff_hbm,
                       deg0_hbm, deg1_hbm,
                       kbuf, dbuf, gbuf, obuf, wbuf, zbuf, deg_sp,
                       sem, semd):
    c = lax.axis_index("c")
    s = lax.axis_index("s")
    wid = s * c32(NC) + c

    @pl.when(s == c32(0))
    def _zero():
        def zrow(k, off):
            zbuf[pl.ds(off, 16)] = jnp.zeros((16,), f32)
            return off + c32(16)
        lax.fori_loop(np.int32(0), np.int32(125), zrow, np.int32(0))
        for k in range(DSP // 2000):
            pltpu.sync_copy(zbuf, deg_sp.at[pl.ds(k * 2000, 2000)])

    plsc.subcore_barrier()

    def group(g, row0):
        row0 = pl.multiple_of(row0, 8)
        pltpu.sync_copy(keys_hbm.at[pl.ds(row0, GA)], kbuf)
        pltpu.sync_copy(dst_hbm.at[pl.ds(row0, GA)], dbuf)
        cps = [
            pltpu.async_copy(table_hbm.at[kbuf.at[c32(j)]], gbuf.at[c32(j)],
                             sem)
            for j in range(GA)
        ]
        for cp in cps:
            cp.wait()
        for j in range(GA):
            rg = row0 + c32(j)
            for q in range(WE // 16):
                sl = pl.ds(q * 16, 16)
                rep = gbuf[j, sl]
                exp = rg * c32(WE) + c32(q * 16) + _iota16()
                keep = rep == exp
                dump = c32(N) + _iota16()
                obuf[j, sl] = jnp.where(keep, dbuf[j, sl], dump)
                wbuf[j, sl] = jnp.where(keep, f32(1.0), f32(0.0))
        pltpu.sync_copy(obuf, dst_eff_hbm.at[pl.ds(row0, GA)])
        dps = [
            pltpu.async_copy(wbuf.at[c32(j)], deg_sp.at[dbuf.at[c32(j)]],
                             semd, add=True)
            for j in range(GA)
        ]
        for cp in dps:
            cp.wait()
        return row0 + c32(GA)

    lax.fori_loop(np.int32(0), np.int32(NGA), group, wid * c32(RPW))

    plsc.subcore_barrier()

    @pl.when(s == c32(0))
    def _readout():
        for k in range(N // 2000):
            pltpu.sync_copy(deg_sp.at[pl.ds(k * 2000, 2000)], zbuf)

            @pl.when(c == c32(0))
            def _w0():
                pltpu.sync_copy(zbuf, deg0_hbm.at[pl.ds(k * 2000, 2000)])

            @pl.when(c == c32(1))
            def _w1():
                pltpu.sync_copy(zbuf, deg1_hbm.at[pl.ds(k * 2000, 2000)])


def _dedup_gather(keys2d, table, dst2d):
    return pl.kernel(
        _dedup_gather_body,
        out_type=[
            jax.ShapeDtypeStruct((ROWSP, WE), i32),
            jax.ShapeDtypeStruct((N,), f32),
            jax.ShapeDtypeStruct((N,), f32),
        ],
        mesh=_mesh,
        scratch_types=[
            pltpu.VMEM((GA, WE), i32),
            pltpu.VMEM((GA, WE), i32),
            pltpu.VMEM((GA, WE), i32),
            pltpu.VMEM((GA, WE), i32),
            pltpu.VMEM((GA, WE), f32),
            pltpu.VMEM((2000,), f32),
            pltpu.VMEM_SHARED((DSP,), f32),
            pltpu.SemaphoreType.DMA,
            pltpu.SemaphoreType.DMA,
        ],
    )(keys2d, table, dst2d)


def _scale_body(dega_ref, degb_ref, x_ref, dinv_ref, y0_ref, y1_ref):
    deg = dega_ref[0, 0, :] + degb_ref[0, 0, :] + f32(1.0)
    dinv = lax.rsqrt(deg)
    dinv_ref[0, 0, :] = dinv
    y = dinv[:, None] * x_ref[...]
    y0_ref[...] = y[:, :HC]
    y1_ref[...] = y[:, HC:]


def _scale_stage(deg0, deg1, x):
    dega3 = deg0.reshape(NB, 1, BN)
    degb3 = deg1.reshape(NB, 1, BN)
    return pl.pallas_call(
        _scale_body,
        grid=(NB,),
        in_specs=[
            pl.BlockSpec((1, 1, BN), lambda i: (i, _Z, _Z)),
            pl.BlockSpec((1, 1, BN), lambda i: (i, _Z, _Z)),
            pl.BlockSpec((BN, CH), lambda i: (i, _Z)),
        ],
        out_specs=[
            pl.BlockSpec((1, 1, BN), lambda i: (i, _Z, _Z)),
            pl.BlockSpec((BN, HC), lambda i: (i, _Z)),
            pl.BlockSpec((BN, HC), lambda i: (i, _Z)),
        ],
        out_shape=[
            jax.ShapeDtypeStruct((NB, 1, BN), f32),
            jax.ShapeDtypeStruct((N, HC), f32),
            jax.ShapeDtypeStruct((N, HC), f32),
        ],
    )(dega3, degb3, x)


def _agg_half(src_hbm, dst_hbm, y_hbm, out2d, acc, sbuf, dbuf, rbuf,
              sem_g, sem_s, s):
    def zrow(k, kk):
        for q in range(HC // 16):
            rbuf[c32(0), kk, pl.ds(q * 16, 16)] = jnp.zeros((16,), f32)
        return kk + c32(1)
    lax.fori_loop(np.int32(0), np.int32(128), zrow, np.int32(0))

    def zblk_full(bi, blk):
        @pl.when(blk < c32(NP // 128))
        def _():
            start = pl.multiple_of(blk * c32(128), 128)
            pltpu.sync_copy(rbuf.at[c32(0)], acc.at[pl.ds(start, 128)])
        return blk + c32(NS)
    lax.fori_loop(np.int32(0), np.int32((NP // 128 + NS - 1) // NS),
                  zblk_full, s)

    @pl.when(s == c32(NS - 1))
    def _ztail():
        t0 = (NP // 128) * 128
        pltpu.sync_copy(rbuf.at[c32(0), pl.ds(0, NP - t0)],
                        acc.at[pl.ds(t0, NP - t0)])

    plsc.subcore_barrier()

    def group(g, row0):
        row0 = pl.multiple_of(row0, 8)
        pltpu.sync_copy(src_hbm.at[pl.ds(row0, GD)], sbuf)
        pltpu.sync_copy(dst_hbm.at[pl.ds(row0, GD)], dbuf)
        gets = [pltpu.async_copy(y_hbm.at[sbuf.at[c32(0)]],
                                 rbuf.at[c32(0)], sem_g)]
        puts = []
        for j in range(GD):
            gets[j].wait()
            puts.append(pltpu.async_copy(rbuf.at[c32(j % 2)],
                                         acc.at[dbuf.at[c32(j)]], sem_s,
                                         add=True))
            if j + 1 < GD:
                if j >= 1:
                    puts[j - 1].wait()
                gets.append(pltpu.async_copy(y_hbm.at[sbuf.at[c32(j + 1)]],
                                             rbuf.at[c32((j + 1) % 2)],
                                             sem_g))
        puts[GD - 2].wait()
        puts[GD - 1].wait()
        return row0 + c32(GD)

    lax.fori_loop(np.int32(0), np.int32(NGD), group, s * c32(RPT))

    plsc.subcore_barrier()

    def wblk(bi, blk):
        @pl.when(blk < c32(N // 128))
        def _():
            start = pl.multiple_of(blk * c32(128), 128)
            pltpu.sync_copy(acc.at[pl.ds(start, 128)], rbuf.at[c32(0)])
            pltpu.sync_copy(rbuf.at[c32(0)], out2d.at[pl.ds(start, 128)])
        return blk + c32(NS)
    lax.fori_loop(np.int32(0), np.int32((N // 128 + NS - 1) // NS),
                  wblk, s)

    @pl.when(s == c32(NS - 1))
    def _wtail():
        t0 = (N // 128) * 128
        sz = N - t0
        pltpu.sync_copy(acc.at[pl.ds(t0, sz)], rbuf.at[c32(0), pl.ds(0, sz)])
        pltpu.sync_copy(rbuf.at[c32(0), pl.ds(0, sz)],
                        out2d.at[pl.ds(t0, sz)])


def _agg_body(src_hbm, dst_hbm, y0_hbm, y1_hbm, agg0_hbm, agg1_hbm,
              acc, sbuf, dbuf, rbuf, sem_g, sem_s):
    c = lax.axis_index("c")
    s = lax.axis_index("s")

    @pl.when(c == c32(0))
    def _half0():
        _agg_half(src_hbm, dst_hbm, y0_hbm, agg0_hbm, acc, sbuf, dbuf,
                  rbuf, sem_g, sem_s, s)

    @pl.when(c == c32(1))
    def _half1():
        _agg_half(src_hbm, dst_hbm, y1_hbm, agg1_hbm, acc, sbuf, dbuf,
                  rbuf, sem_g, sem_s, s)


def _agg_stage(src2d, dst_eff2d, y0, y1):
    return pl.kernel(
        _agg_body,
        out_type=[
            jax.ShapeDtypeStruct((N, HC), f32),
            jax.ShapeDtypeStruct((N, HC), f32),
        ],
        mesh=_mesh,
        scratch_types=[
            pltpu.VMEM_SHARED((NP, HC), f32),
            pltpu.VMEM((GD, WE), i32),
            pltpu.VMEM((GD, WE), i32),
            pltpu.VMEM((2, WE, HC), f32),
            pltpu.SemaphoreType.DMA,
            pltpu.SemaphoreType.DMA,
        ],
    )(src2d, dst_eff2d, y0, y1)


def _dense_body(x_ref, agg0_ref, agg1_ref, dinv_ref, batch_ref, w1_ref,
                out_ref, acc_ref, cnt_ref):
    i = pl.program_id(0)

    @pl.when(i == 0)
    def _init():
        acc_ref[...] = jnp.zeros_like(acc_ref)
        cnt_ref[...] = jnp.zeros_like(cnt_ref)

    dinv = dinv_ref[0, 0, :]
    agg = jnp.concatenate([agg0_ref[...], agg1_ref[...]], axis=1)
    xb = x_ref[...]
    d = dinv[:, None]
    a = d * agg + (d * d) * xb
    h = (1.0 - ALPHA) * a + ALPHA * xb
    r = lax.broadcasted_iota(i32, (CH, CH), 0)
    cc = lax.broadcasted_iota(i32, (CH, CH), 1)
    w1p = (f32(BETA) * w1_ref[...]
           + jnp.where(r == cc, f32(1.0 - BETA), f32(0.0)))
    t = jnp.dot(h, w1p, preferred_element_type=f32)
    o = jnp.maximum(t, 0.0)

    b = batch_ref[0, 0, :]
    onehot = (b[:, None] == lax.broadcasted_iota(i32, (BN, G), 1))
    onehot = onehot.astype(f32)
    acc_ref[...] += lax.dot_general(
        onehot, o, (((0,), (0,)), ((), ())), preferred_element_type=f32)
    cnt_ref[...] += jnp.sum(onehot, axis=0, keepdims=True)

    @pl.when(i == NB - 1)
    def _fin():
        cnts = jnp.maximum(cnt_ref[...], 1.0)
        out_ref[...] = acc_ref[...] / cnts[0, :][:, None]


def _dense_stage(x, agg0, agg1, dinv3, batch_i32, W1):
    batch3 = batch_i32.reshape(NB, 1, BN)
    return pl.pallas_call(
        _dense_body,
        grid=(NB,),
        in_specs=[
            pl.BlockSpec((BN, CH), lambda i: (i, _Z)),
            pl.BlockSpec((BN, HC), lambda i: (i, _Z)),
            pl.BlockSpec((BN, HC), lambda i: (i, _Z)),
            pl.BlockSpec((1, 1, BN), lambda i: (i, _Z, _Z)),
            pl.BlockSpec((1, 1, BN), lambda i: (i, _Z, _Z)),
            pl.BlockSpec((CH, CH), lambda i: (_Z, _Z)),
        ],
        out_specs=pl.BlockSpec((G, CH), lambda i: (_Z, _Z)),
        out_shape=jax.ShapeDtypeStruct((G, CH), f32),
        scratch_shapes=[
            pltpu.VMEM((G, CH), f32),
            pltpu.VMEM((1, G), f32),
        ],
    )(x, agg0, agg1, dinv3, batch3, W1)


def kernel(x, edge_index, batch, W1):
    x = x.astype(f32)
    ei = edge_index.astype(i32)
    src2 = jnp.concatenate([ei[0], ei[1]])
    dst2 = jnp.concatenate([ei[1], ei[0]])
    batch_i32 = batch.astype(i32)

    pad = E2P - E2
    pidx = jnp.arange(pad, dtype=i32)
    src2p = jnp.concatenate([src2, pidx % N])
    dst2p = jnp.concatenate([dst2, N + (pidx % 16)])
    keysp = jnp.concatenate([src2 * N + dst2, TBL + (pidx % 16)])

    keys2d = keysp.reshape(ROWSP, WE)
    src2d = src2p.reshape(ROWSP, WE)
    dst2d = dst2p.reshape(ROWSP, WE)

    table = _dedup_scatter(keys2d)
    dst_eff2d, deg0, deg1 = _dedup_gather(keys2d, table, dst2d)
    dinv3, y0, y1 = _scale_stage(deg0, deg1, x)
    agg0, agg1 = _agg_stage(src2d, dst_eff2d, y0, y1)
    return _dense_stage(x, agg0, agg1, dinv3, batch_i32, W1)


# speedup vs baseline: 3.1656x; 3.1656x over previous
import math

import jax
import jax.numpy as jnp
import numpy as np
from jax import lax
from jax.experimental import pallas as pl
from jax.experimental.pallas import tpu as pltpu
from jax.experimental.pallas import tpu_sc as plsc

N = 10000
E = 160000
E2 = 2 * E
CH = 256
HC = CH // 2
G = 64
ALPHA = 0.1
BETA = math.log(0.5 / 8 + 1.0)
TBL = N * N

NC = 2
NS = 16
NW = NC * NS
WE = 128
ROWSP = 2560
E2P = ROWSP * WE
RPW = ROWSP // NW
RPT = ROWSP // NS
GA = 8
NGA = RPW // GA
GD = 8
NGD = RPT // GD
NP = 10016
DSP = 12000

NB = 10
_Z = np.int32(0)
BN = N // NB

_mesh = plsc.VectorSubcoreMesh(core_axis_name="c", subcore_axis_name="s")

f32 = jnp.float32
i32 = jnp.int32


def c32(v):
    return np.int32(v)


def _iota16():
    return lax.iota(i32, 16)


def _dedup_scatter_body(keys_hbm, ids_hbm, table_hbm, kbuf, ibuf, sem):
    c = lax.axis_index("c")
    s = lax.axis_index("s")
    wid = s * c32(NC) + c
    row0w = pl.multiple_of(wid * c32(RPW), 8)

    pltpu.sync_copy(keys_hbm.at[pl.ds(row0w, RPW)], kbuf)

    def group(g, lr):
        pltpu.sync_copy(ids_hbm.at[pl.ds(row0w + lr, GA)], ibuf)
        cps = [
            pltpu.async_copy(ibuf.at[c32(j)],
                             table_hbm.at[kbuf.at[lr + c32(j)]], sem)
            for j in range(GA)
        ]
        for cp in cps:
            cp.wait()
        return lr + c32(GA)

    lax.fori_loop(np.int32(0), np.int32(NGA), group, np.int32(0))


def _dedup_scatter(keys2d, ids3):
    return pl.kernel(
        _dedup_scatter_body,
        out_type=jax.ShapeDtypeStruct((TBL + 16, 16), i32),
        mesh=_mesh,
        compiler_params=pltpu.CompilerParams(use_tc_tiling_on_sc=False),
        scratch_types=[
            pltpu.VMEM((RPW, WE), i32),
            pltpu.VMEM((GA, WE, 16), i32),
            pltpu.SemaphoreType.DMA,
        ],
    )(keys2d, ids3)


def _dedup_gather_body(keys_hbm, table_hbm, dst_hbm, dst_eff_hbm,
                       deg0_hbm, deg1_hbm,
                       kbuf, dbuf, gbuf, obuf, wbuf, zbuf, deg_sp,
                       sem, semd):
    c = lax.axis_index("c")
    s = lax.axis_index("s")
    wid = s * c32(NC) + c

    @pl.when(s == c32(0))
    def _zero():
        def zrow(k, off):
            zbuf[pl.ds(off, 16)] = jnp.zeros((16,), f32)
            return off + c32(16)
        lax.fori_loop(np.int32(0), np.int32(125), zrow, np.int32(0))
        for k in range(DSP // 2000):
            pltpu.sync_copy(zbuf, deg_sp.at[pl.ds(k * 2000, 2000)])

    plsc.subcore_barrier()

    def group(g, row0):
        row0 = pl.multiple_of(row0, 8)
        pltpu.sync_copy(keys_hbm.at[pl.ds(row0, GA)], kbuf)
        pltpu.sync_copy(dst_hbm.at[pl.ds(row0, GA)], dbuf)
        cps = [
            pltpu.async_copy(table_hbm.at[kbuf.at[c32(j)]], gbuf.at[c32(j)],
                             sem)
            for j in range(GA)
        ]
        for cp in cps:
            cp.wait()
        for j in range(GA):
            rg = row0 + c32(j)
            for q in range(WE // 16):
                sl = pl.ds(q * 16, 16)
                rep = gbuf[j, sl]
                exp = rg * c32(WE) + c32(q * 16) + _iota16()
                keep = rep == exp
                dump = c32(N) + _iota16()
                obuf[j, sl] = jnp.where(keep, dbuf[j, sl], dump)
                wbuf[j, sl] = jnp.where(keep, f32(1.0), f32(0.0))
        pltpu.sync_copy(obuf, dst_eff_hbm.at[pl.ds(row0, GA)])
        dps = [
            pltpu.async_copy(wbuf.at[c32(j)], deg_sp.at[dbuf.at[c32(j)]],
                             semd, add=True)
            for j in range(GA)
        ]
        for cp in dps:
            cp.wait()
        return row0 + c32(GA)

    lax.fori_loop(np.int32(0), np.int32(NGA), group, wid * c32(RPW))

    plsc.subcore_barrier()

    @pl.when(s == c32(0))
    def _readout():
        for k in range(N // 2000):
            pltpu.sync_copy(deg_sp.at[pl.ds(k * 2000, 2000)], zbuf)

            @pl.when(c == c32(0))
            def _w0():
                pltpu.sync_copy(zbuf, deg0_hbm.at[pl.ds(k * 2000, 2000)])

            @pl.when(c == c32(1))
            def _w1():
                pltpu.sync_copy(zbuf, deg1_hbm.at[pl.ds(k * 2000, 2000)])


def _dedup_gather(keys2d, table, dst2d):
    return pl.kernel(
        _dedup_gather_body,
        out_type=[
            jax.ShapeDtypeStruct((ROWSP, WE), i32),
            jax.ShapeDtypeStruct((N,), f32),
            jax.ShapeDtypeStruct((N,), f32),
        ],
        mesh=_mesh,
        scratch_types=[
            pltpu.VMEM((GA, WE), i32),
            pltpu.VMEM((GA, WE), i32),
            pltpu.VMEM((GA, WE), i32),
            pltpu.VMEM((GA, WE), i32),
            pltpu.VMEM((GA, WE), f32),
            pltpu.VMEM((2000,), f32),
            pltpu.VMEM_SHARED((DSP,), f32),
            pltpu.SemaphoreType.DMA,
            pltpu.SemaphoreType.DMA,
        ],
    )(keys2d, table, dst2d)


def _scale_body(dega_ref, degb_ref, x_ref, dinv_ref, y0_ref, y1_ref):
    deg = dega_ref[0, 0, :] + degb_ref[0, 0, :] + f32(1.0)
    dinv = lax.rsqrt(deg)
    dinv_ref[0, 0, :] = dinv
    y = dinv[:, None] * x_ref[...]
    y0_ref[...] = y[:, :HC]
    y1_ref[...] = y[:, HC:]


def _scale_stage(deg0, deg1, x):
    dega3 = deg0.reshape(NB, 1, BN)
    degb3 = deg1.reshape(NB, 1, BN)
    return pl.pallas_call(
        _scale_body,
        grid=(NB,),
        in_specs=[
            pl.BlockSpec((1, 1, BN), lambda i: (i, _Z, _Z)),
            pl.BlockSpec((1, 1, BN), lambda i: (i, _Z, _Z)),
            pl.BlockSpec((BN, CH), lambda i: (i, _Z)),
        ],
        out_specs=[
            pl.BlockSpec((1, 1, BN), lambda i: (i, _Z, _Z)),
            pl.BlockSpec((BN, HC), lambda i: (i, _Z)),
            pl.BlockSpec((BN, HC), lambda i: (i, _Z)),
        ],
        out_shape=[
            jax.ShapeDtypeStruct((NB, 1, BN), f32),
            jax.ShapeDtypeStruct((N, HC), f32),
            jax.ShapeDtypeStruct((N, HC), f32),
        ],
    )(dega3, degb3, x)


def _agg_half(src_hbm, dst_hbm, y_hbm, out2d, acc, sbuf, dbuf, rbuf,
              sem_g, sem_s, s):
    def zrow(k, kk):
        for q in range(HC // 16):
            rbuf[c32(0), kk, pl.ds(q * 16, 16)] = jnp.zeros((16,), f32)
        return kk + c32(1)
    lax.fori_loop(np.int32(0), np.int32(128), zrow, np.int32(0))

    def zblk_full(bi, blk):
        @pl.when(blk < c32(NP // 128))
        def _():
            start = pl.multiple_of(blk * c32(128), 128)
            pltpu.sync_copy(rbuf.at[c32(0)], acc.at[pl.ds(start, 128)])
        return blk + c32(NS)
    lax.fori_loop(np.int32(0), np.int32((NP // 128 + NS - 1) // NS),
                  zblk_full, s)

    @pl.when(s == c32(NS - 1))
    def _ztail():
        t0 = (NP // 128) * 128
        pltpu.sync_copy(rbuf.at[c32(0), pl.ds(0, NP - t0)],
                        acc.at[pl.ds(t0, NP - t0)])

    plsc.subcore_barrier()

    def group(g, row0):
        row0 = pl.multiple_of(row0, 8)
        pltpu.sync_copy(src_hbm.at[pl.ds(row0, GD)], sbuf)
        pltpu.sync_copy(dst_hbm.at[pl.ds(row0, GD)], dbuf)
        gets = [pltpu.async_copy(y_hbm.at[sbuf.at[c32(0)]],
                                 rbuf.at[c32(0)], sem_g)]
        puts = []
        for j in range(GD):
            gets[j].wait()
            puts.append(pltpu.async_copy(rbuf.at[c32(j % 2)],
                                         acc.at[dbuf.at[c32(j)]], sem_s,
                                         add=True))
            if j + 1 < GD:
                if j >= 1:
                    puts[j - 1].wait()
                gets.append(pltpu.async_copy(y_hbm.at[sbuf.at[c32(j + 1)]],
                                             rbuf.at[c32((j + 1) % 2)],
                                             sem_g))
        puts[GD - 2].wait()
        puts[GD - 1].wait()
        return row0 + c32(GD)

    lax.fori_loop(np.int32(0), np.int32(NGD), group, s * c32(RPT))

    plsc.subcore_barrier()

    def wblk(bi, blk):
        @pl.when(blk < c32(N // 128))
        def _():
            start = pl.multiple_of(blk * c32(128), 128)
            pltpu.sync_copy(acc.at[pl.ds(start, 128)], rbuf.at[c32(0)])
            pltpu.sync_copy(rbuf.at[c32(0)], out2d.at[pl.ds(start, 128)])
        return blk + c32(NS)
    lax.fori_loop(np.int32(0), np.int32((N // 128 + NS - 1) // NS),
                  wblk, s)

    @pl.when(s == c32(NS - 1))
    def _wtail():
        t0 = (N // 128) * 128
        sz = N - t0
        pltpu.sync_copy(acc.at[pl.ds(t0, sz)], rbuf.at[c32(0), pl.ds(0, sz)])
        pltpu.sync_copy(rbuf.at[c32(0), pl.ds(0, sz)],
                        out2d.at[pl.ds(t0, sz)])


def _agg_body(src_hbm, dst_hbm, y0_hbm, y1_hbm, agg0_hbm, agg1_hbm,
              acc, sbuf, dbuf, rbuf, sem_g, sem_s):
    c = lax.axis_index("c")
    s = lax.axis_index("s")

    @pl.when(c == c32(0))
    def _half0():
        _agg_half(src_hbm, dst_hbm, y0_hbm, agg0_hbm, acc, sbuf, dbuf,
                  rbuf, sem_g, sem_s, s)

    @pl.when(c == c32(1))
    def _half1():
        _agg_half(src_hbm, dst_hbm, y1_hbm, agg1_hbm, acc, sbuf, dbuf,
                  rbuf, sem_g, sem_s, s)


def _agg_stage(src2d, dst_eff2d, y0, y1):
    return pl.kernel(
        _agg_body,
        out_type=[
            jax.ShapeDtypeStruct((N, HC), f32),
            jax.ShapeDtypeStruct((N, HC), f32),
        ],
        mesh=_mesh,
        scratch_types=[
            pltpu.VMEM_SHARED((NP, HC), f32),
            pltpu.VMEM((GD, WE), i32),
            pltpu.VMEM((GD, WE), i32),
            pltpu.VMEM((2, WE, HC), f32),
            pltpu.SemaphoreType.DMA,
            pltpu.SemaphoreType.DMA,
        ],
    )(src2d, dst_eff2d, y0, y1)


def _dense_body(x_ref, agg0_ref, agg1_ref, dinv_ref, batch_ref, w1_ref,
                out_ref, acc_ref, cnt_ref):
    i = pl.program_id(0)

    @pl.when(i == 0)
    def _init():
        acc_ref[...] = jnp.zeros_like(acc_ref)
        cnt_ref[...] = jnp.zeros_like(cnt_ref)

    dinv = dinv_ref[0, 0, :]
    agg = jnp.concatenate([agg0_ref[...], agg1_ref[...]], axis=1)
    xb = x_ref[...]
    d = dinv[:, None]
    a = d * agg + (d * d) * xb
    h = (1.0 - ALPHA) * a + ALPHA * xb
    r = lax.broadcasted_iota(i32, (CH, CH), 0)
    cc = lax.broadcasted_iota(i32, (CH, CH), 1)
    w1p = (f32(BETA) * w1_ref[...]
           + jnp.where(r == cc, f32(1.0 - BETA), f32(0.0)))
    t = jnp.dot(h, w1p, preferred_element_type=f32)
    o = jnp.maximum(t, 0.0)

    b = batch_ref[0, 0, :]
    onehot = (b[:, None] == lax.broadcasted_iota(i32, (BN, G), 1))
    onehot = onehot.astype(f32)
    acc_ref[...] += lax.dot_general(
        onehot, o, (((0,), (0,)), ((), ())), preferred_element_type=f32)
    cnt_ref[...] += jnp.sum(onehot, axis=0, keepdims=True)

    @pl.when(i == NB - 1)
    def _fin():
        cnts = jnp.maximum(cnt_ref[...], 1.0)
        out_ref[...] = acc_ref[...] / cnts[0, :][:, None]


def _dense_stage(x, agg0, agg1, dinv3, batch_i32, W1):
    batch3 = batch_i32.reshape(NB, 1, BN)
    return pl.pallas_call(
        _dense_body,
        grid=(NB,),
        in_specs=[
            pl.BlockSpec((BN, CH), lambda i: (i, _Z)),
            pl.BlockSpec((BN, HC), lambda i: (i, _Z)),
            pl.BlockSpec((BN, HC), lambda i: (i, _Z)),
            pl.BlockSpec((1, 1, BN), lambda i: (i, _Z, _Z)),
            pl.BlockSpec((1, 1, BN), lambda i: (i, _Z, _Z)),
            pl.BlockSpec((CH, CH), lambda i: (_Z, _Z)),
        ],
        out_specs=pl.BlockSpec((G, CH), lambda i: (_Z, _Z)),
        out_shape=jax.ShapeDtypeStruct((G, CH), f32),
        scratch_shapes=[
            pltpu.VMEM((G, CH), f32),
            pltpu.VMEM((1, G), f32),
        ],
    )(x, agg0, agg1, dinv3, batch3, W1)


def kernel(x, edge_index, batch, W1):
    x = x.astype(f32)
    ei = edge_index.astype(i32)
    src2 = jnp.concatenate([ei[0], ei[1]])
    dst2 = jnp.concatenate([ei[1], ei[0]])
    batch_i32 = batch.astype(i32)

    pad = E2P - E2
    pidx = jnp.arange(pad, dtype=i32)
    src2p = jnp.concatenate([src2, pidx % N])
    dst2p = jnp.concatenate([dst2, N + (pidx % 16)])
    keysp = jnp.concatenate([src2 * N + dst2, TBL + (pidx % 16)])

    keys2d = keysp.reshape(ROWSP, WE)
    keys16_2d = (keysp * 16).reshape(ROWSP, WE)
    src2d = src2p.reshape(ROWSP, WE)
    dst2d = dst2p.reshape(ROWSP, WE)
    ids3 = jnp.broadcast_to(
        jnp.arange(E2P, dtype=i32).reshape(ROWSP, WE)[:, :, None],
        (ROWSP, WE, 16))

    table = _dedup_scatter(keys2d, ids3)
    dst_eff2d, deg0, deg1 = _dedup_gather(keys16_2d, table.reshape(-1),
                                          dst2d)
    dinv3, y0, y1 = _scale_stage(deg0, deg1, x)
    agg0, agg1 = _agg_stage(src2d, dst_eff2d, y0, y1)
    return _dense_stage(x, agg0, agg1, dinv3, batch_i32, W1)


# speedup vs baseline: 3.2718x; 1.0336x over previous
import math

import jax
import jax.numpy as jnp
import numpy as np
from jax import lax
from jax.experimental import pallas as pl
from jax.experimental.pallas import tpu as pltpu
from jax.experimental.pallas import tpu_sc as plsc

N = 10000
E = 160000
E2 = 2 * E
CH = 256
HC = CH // 2
G = 64
ALPHA = 0.1
BETA = math.log(0.5 / 8 + 1.0)
TBL = N * N

NC = 2
NS = 16
NW = NC * NS
WE = 128
ROWSP = 2560
E2P = ROWSP * WE
RPW = ROWSP // NW
RPT = ROWSP // NS
GA = 8
NGA = RPW // GA
GD = 16
NGD = RPT // GD
NP = 10016
DSP = 12000

NB = 10
_Z = np.int32(0)
BN = N // NB

_mesh = plsc.VectorSubcoreMesh(core_axis_name="c", subcore_axis_name="s")

f32 = jnp.float32
i32 = jnp.int32


def c32(v):
    return np.int32(v)


def _iota16():
    return lax.iota(i32, 16)


def _dedup_scatter_body(keys_hbm, ids_hbm, table_hbm, kbuf, ibuf, sem):
    c = lax.axis_index("c")
    s = lax.axis_index("s")
    wid = s * c32(NC) + c
    row0w = pl.multiple_of(wid * c32(RPW), 8)

    pltpu.sync_copy(keys_hbm.at[pl.ds(row0w, RPW)], kbuf)

    def group(g, lr):
        pltpu.sync_copy(ids_hbm.at[pl.ds(row0w + lr, GA)], ibuf)
        cps = [
            pltpu.async_copy(ibuf.at[c32(j)],
                             table_hbm.at[kbuf.at[lr + c32(j)]], sem)
            for j in range(GA)
        ]
        for cp in cps:
            cp.wait()
        return lr + c32(GA)

    lax.fori_loop(np.int32(0), np.int32(NGA), group, np.int32(0))


def _dedup_scatter(keys2d, ids3):
    return pl.kernel(
        _dedup_scatter_body,
        out_type=jax.ShapeDtypeStruct((TBL + 16, 16), i32),
        mesh=_mesh,
        compiler_params=pltpu.CompilerParams(use_tc_tiling_on_sc=False),
        scratch_types=[
            pltpu.VMEM((RPW, WE), i32),
            pltpu.VMEM((GA, WE, 16), i32),
            pltpu.SemaphoreType.DMA,
        ],
    )(keys2d, ids3)


def _dedup_gather_body(keys_hbm, table_hbm, dst_hbm, dst_eff_hbm,
                       deg0_hbm, deg1_hbm,
                       kbuf, dbuf, gbuf, obuf, wbuf, zbuf, deg_sp,
                       sem, semd):
    c = lax.axis_index("c")
    s = lax.axis_index("s")
    wid = s * c32(NC) + c

    @pl.when(s == c32(0))
    def _zero():
        def zrow(k, off):
            zbuf[pl.ds(off, 16)] = jnp.zeros((16,), f32)
            return off + c32(16)
        lax.fori_loop(np.int32(0), np.int32(125), zrow, np.int32(0))
        for k in range(DSP // 2000):
            pltpu.sync_copy(zbuf, deg_sp.at[pl.ds(k * 2000, 2000)])

    plsc.subcore_barrier()

    def group(g, row0):
        row0 = pl.multiple_of(row0, 8)
        pltpu.sync_copy(keys_hbm.at[pl.ds(row0, GA)], kbuf)
        pltpu.sync_copy(dst_hbm.at[pl.ds(row0, GA)], dbuf)
        cps = [
            pltpu.async_copy(table_hbm.at[kbuf.at[c32(j)]], gbuf.at[c32(j)],
                             sem)
            for j in range(GA)
        ]
        for cp in cps:
            cp.wait()
        for j in range(GA):
            rg = row0 + c32(j)
            for q in range(WE // 16):
                sl = pl.ds(q * 16, 16)
                rep = gbuf[j, sl]
                exp = rg * c32(WE) + c32(q * 16) + _iota16()
                keep = rep == exp
                dump = c32(N) + _iota16()
                obuf[j, sl] = jnp.where(keep, dbuf[j, sl], dump)
                wbuf[j, sl] = jnp.where(keep, f32(1.0), f32(0.0))
        pltpu.sync_copy(obuf, dst_eff_hbm.at[pl.ds(row0, GA)])
        dps = [
            pltpu.async_copy(wbuf.at[c32(j)], deg_sp.at[dbuf.at[c32(j)]],
                             semd, add=True)
            for j in range(GA)
        ]
        for cp in dps:
            cp.wait()
        return row0 + c32(GA)

    lax.fori_loop(np.int32(0), np.int32(NGA), group, wid * c32(RPW))

    plsc.subcore_barrier()

    @pl.when(s == c32(0))
    def _readout():
        for k in range(N // 2000):
            pltpu.sync_copy(deg_sp.at[pl.ds(k * 2000, 2000)], zbuf)

            @pl.when(c == c32(0))
            def _w0():
                pltpu.sync_copy(zbuf, deg0_hbm.at[pl.ds(k * 2000, 2000)])

            @pl.when(c == c32(1))
            def _w1():
                pltpu.sync_copy(zbuf, deg1_hbm.at[pl.ds(k * 2000, 2000)])


def _dedup_gather(keys2d, table, dst2d):
    return pl.kernel(
        _dedup_gather_body,
        out_type=[
            jax.ShapeDtypeStruct((ROWSP, WE), i32),
            jax.ShapeDtypeStruct((N,), f32),
            jax.ShapeDtypeStruct((N,), f32),
        ],
        mesh=_mesh,
        scratch_types=[
            pltpu.VMEM((GA, WE), i32),
            pltpu.VMEM((GA, WE), i32),
            pltpu.VMEM((GA, WE), i32),
            pltpu.VMEM((GA, WE), i32),
            pltpu.VMEM((GA, WE), f32),
            pltpu.VMEM((2000,), f32),
            pltpu.VMEM_SHARED((DSP,), f32),
            pltpu.SemaphoreType.DMA,
            pltpu.SemaphoreType.DMA,
        ],
    )(keys2d, table, dst2d)


def _scale_body(dega_ref, degb_ref, x_ref, dinv_ref, y0_ref, y1_ref):
    deg = dega_ref[0, 0, :] + degb_ref[0, 0, :] + f32(1.0)
    dinv = lax.rsqrt(deg)
    dinv_ref[0, 0, :] = dinv
    y = dinv[:, None] * x_ref[...]
    y0_ref[...] = y[:, :HC]
    y1_ref[...] = y[:, HC:]


def _scale_stage(deg0, deg1, x):
    dega3 = deg0.reshape(NB, 1, BN)
    degb3 = deg1.reshape(NB, 1, BN)
    return pl.pallas_call(
        _scale_body,
        grid=(NB,),
        in_specs=[
            pl.BlockSpec((1, 1, BN), lambda i: (i, _Z, _Z)),
            pl.BlockSpec((1, 1, BN), lambda i: (i, _Z, _Z)),
            pl.BlockSpec((BN, CH), lambda i: (i, _Z)),
        ],
        out_specs=[
            pl.BlockSpec((1, 1, BN), lambda i: (i, _Z, _Z)),
            pl.BlockSpec((BN, HC), lambda i: (i, _Z)),
            pl.BlockSpec((BN, HC), lambda i: (i, _Z)),
        ],
        out_shape=[
            jax.ShapeDtypeStruct((NB, 1, BN), f32),
            jax.ShapeDtypeStruct((N, HC), f32),
            jax.ShapeDtypeStruct((N, HC), f32),
        ],
    )(dega3, degb3, x)


def _agg_half(src_hbm, dst_hbm, y_hbm, out2d, acc, sbuf, dbuf, rbuf,
              sem_i, sem_g, sem_s, s):
    def zrow(k, kk):
        for q in range(HC // 16):
            rbuf[c32(0), kk, pl.ds(q * 16, 16)] = jnp.zeros((16,), f32)
        return kk + c32(1)
    lax.fori_loop(np.int32(0), np.int32(128), zrow, np.int32(0))

    def zblk_full(bi, blk):
        @pl.when(blk < c32(NP // 128))
        def _():
            start = pl.multiple_of(blk * c32(128), 128)
            pltpu.sync_copy(rbuf.at[c32(0)], acc.at[pl.ds(start, 128)])
        return blk + c32(NS)
    lax.fori_loop(np.int32(0), np.int32((NP // 128 + NS - 1) // NS),
                  zblk_full, s)

    @pl.when(s == c32(NS - 1))
    def _ztail():
        t0 = (NP // 128) * 128
        pltpu.sync_copy(rbuf.at[c32(0), pl.ds(0, NP - t0)],
                        acc.at[pl.ds(t0, NP - t0)])

    plsc.subcore_barrier()

    def group(g, row0):
        row0 = pl.multiple_of(row0, 8)
        cs = pltpu.async_copy(src_hbm.at[pl.ds(row0, GD)], sbuf, sem_i)
        cd = pltpu.async_copy(dst_hbm.at[pl.ds(row0, GD)], dbuf, sem_i)
        cs.wait()
        cd.wait()
        gets = [pltpu.async_copy(y_hbm.at[sbuf.at[c32(0)]],
                                 rbuf.at[c32(0)], sem_g)]
        puts = []
        for j in range(GD):
            gets[j].wait()
            puts.append(pltpu.async_copy(rbuf.at[c32(j % 2)],
                                         acc.at[dbuf.at[c32(j)]], sem_s,
                                         add=True))
            if j + 1 < GD:
                if j >= 1:
                    puts[j - 1].wait()
                gets.append(pltpu.async_copy(y_hbm.at[sbuf.at[c32(j + 1)]],
                                             rbuf.at[c32((j + 1) % 2)],
                                             sem_g))
        puts[GD - 2].wait()
        puts[GD - 1].wait()
        return row0 + c32(GD)

    lax.fori_loop(np.int32(0), np.int32(NGD), group, s * c32(RPT))

    plsc.subcore_barrier()

    def wblk(bi, blk):
        @pl.when(blk < c32(N // 128))
        def _():
            start = pl.multiple_of(blk * c32(128), 128)
            pltpu.sync_copy(acc.at[pl.ds(start, 128)], rbuf.at[c32(0)])
            pltpu.sync_copy(rbuf.at[c32(0)], out2d.at[pl.ds(start, 128)])
        return blk + c32(NS)
    lax.fori_loop(np.int32(0), np.int32((N // 128 + NS - 1) // NS),
                  wblk, s)

    @pl.when(s == c32(NS - 1))
    def _wtail():
        t0 = (N // 128) * 128
        sz = N - t0
        pltpu.sync_copy(acc.at[pl.ds(t0, sz)], rbuf.at[c32(0), pl.ds(0, sz)])
        pltpu.sync_copy(rbuf.at[c32(0), pl.ds(0, sz)],
                        out2d.at[pl.ds(t0, sz)])


def _agg_body(src_hbm, dst_hbm, y0_hbm, y1_hbm, agg0_hbm, agg1_hbm,
              acc, sbuf, dbuf, rbuf, sem_i, sem_g, sem_s):
    c = lax.axis_index("c")
    s = lax.axis_index("s")

    @pl.when(c == c32(0))
    def _half0():
        _agg_half(src_hbm, dst_hbm, y0_hbm, agg0_hbm, acc, sbuf, dbuf,
                  rbuf, sem_i, sem_g, sem_s, s)

    @pl.when(c == c32(1))
    def _half1():
        _agg_half(src_hbm, dst_hbm, y1_hbm, agg1_hbm, acc, sbuf, dbuf,
                  rbuf, sem_i, sem_g, sem_s, s)


def _agg_stage(src2d, dst_eff2d, y0, y1):
    return pl.kernel(
        _agg_body,
        out_type=[
            jax.ShapeDtypeStruct((N, HC), f32),
            jax.ShapeDtypeStruct((N, HC), f32),
        ],
        mesh=_mesh,
        scratch_types=[
            pltpu.VMEM_SHARED((NP, HC), f32),
            pltpu.VMEM((GD, WE), i32),
            pltpu.VMEM((GD, WE), i32),
            pltpu.VMEM((2, WE, HC), f32),
            pltpu.SemaphoreType.DMA,
            pltpu.SemaphoreType.DMA,
            pltpu.SemaphoreType.DMA,
        ],
    )(src2d, dst_eff2d, y0, y1)


def _dense_body(x_ref, agg0_ref, agg1_ref, dinv_ref, batch_ref, w1_ref,
                out_ref, acc_ref, cnt_ref):
    i = pl.program_id(0)

    @pl.when(i == 0)
    def _init():
        acc_ref[...] = jnp.zeros_like(acc_ref)
        cnt_ref[...] = jnp.zeros_like(cnt_ref)

    dinv = dinv_ref[0, 0, :]
    agg = jnp.concatenate([agg0_ref[...], agg1_ref[...]], axis=1)
    xb = x_ref[...]
    d = dinv[:, None]
    a = d * agg + (d * d) * xb
    h = (1.0 - ALPHA) * a + ALPHA * xb
    r = lax.broadcasted_iota(i32, (CH, CH), 0)
    cc = lax.broadcasted_iota(i32, (CH, CH), 1)
    w1p = (f32(BETA) * w1_ref[...]
           + jnp.where(r == cc, f32(1.0 - BETA), f32(0.0)))
    t = jnp.dot(h, w1p, preferred_element_type=f32)
    o = jnp.maximum(t, 0.0)

    b = batch_ref[0, 0, :]
    onehot = (b[:, None] == lax.broadcasted_iota(i32, (BN, G), 1))
    onehot = onehot.astype(f32)
    acc_ref[...] += lax.dot_general(
        onehot, o, (((0,), (0,)), ((), ())), preferred_element_type=f32)
    cnt_ref[...] += jnp.sum(onehot, axis=0, keepdims=True)

    @pl.when(i == NB - 1)
    def _fin():
        cnts = jnp.maximum(cnt_ref[...], 1.0)
        out_ref[...] = acc_ref[...] / cnts[0, :][:, None]


def _dense_stage(x, agg0, agg1, dinv3, batch_i32, W1):
    batch3 = batch_i32.reshape(NB, 1, BN)
    return pl.pallas_call(
        _dense_body,
        grid=(NB,),
        in_specs=[
            pl.BlockSpec((BN, CH), lambda i: (i, _Z)),
            pl.BlockSpec((BN, HC), lambda i: (i, _Z)),
            pl.BlockSpec((BN, HC), lambda i: (i, _Z)),
            pl.BlockSpec((1, 1, BN), lambda i: (i, _Z, _Z)),
            pl.BlockSpec((1, 1, BN), lambda i: (i, _Z, _Z)),
            pl.BlockSpec((CH, CH), lambda i: (_Z, _Z)),
        ],
        out_specs=pl.BlockSpec((G, CH), lambda i: (_Z, _Z)),
        out_shape=jax.ShapeDtypeStruct((G, CH), f32),
        scratch_shapes=[
            pltpu.VMEM((G, CH), f32),
            pltpu.VMEM((1, G), f32),
        ],
    )(x, agg0, agg1, dinv3, batch3, W1)


def kernel(x, edge_index, batch, W1):
    x = x.astype(f32)
    ei = edge_index.astype(i32)
    src2 = jnp.concatenate([ei[0], ei[1]])
    dst2 = jnp.concatenate([ei[1], ei[0]])
    batch_i32 = batch.astype(i32)

    pad = E2P - E2
    pidx = jnp.arange(pad, dtype=i32)
    src2p = jnp.concatenate([src2, pidx % N])
    dst2p = jnp.concatenate([dst2, N + (pidx % 16)])
    keysp = jnp.concatenate([src2 * N + dst2, TBL + (pidx % 16)])

    keys2d = keysp.reshape(ROWSP, WE)
    keys16_2d = (keysp * 16).reshape(ROWSP, WE)
    src2d = src2p.reshape(ROWSP, WE)
    dst2d = dst2p.reshape(ROWSP, WE)
    ids3 = jnp.broadcast_to(
        jnp.arange(E2P, dtype=i32).reshape(ROWSP, WE)[:, :, None],
        (ROWSP, WE, 16))

    table = _dedup_scatter(keys2d, ids3)
    dst_eff2d, deg0, deg1 = _dedup_gather(keys16_2d, table.reshape(-1),
                                          dst2d)
    dinv3, y0, y1 = _scale_stage(deg0, deg1, x)
    agg0, agg1 = _agg_stage(src2d, dst_eff2d, y0, y1)
    return _dense_stage(x, agg0, agg1, dinv3, batch_i32, W1)


# speedup vs baseline: 3.2848x; 1.0040x over previous
import math

import jax
import jax.numpy as jnp
import numpy as np
from jax import lax
from jax.experimental import pallas as pl
from jax.experimental.pallas import tpu as pltpu
from jax.experimental.pallas import tpu_sc as plsc

N = 10000
E = 160000
E2 = 2 * E
CH = 256
HC = CH // 2
G = 64
ALPHA = 0.1
BETA = math.log(0.5 / 8 + 1.0)
TBL = N * N

NC = 2
NS = 16
NW = NC * NS
WE = 128
ROWSP = 2560
E2P = ROWSP * WE
RPW = ROWSP // NW
RPT = ROWSP // NS
GA = 8
NGA = RPW // GA
GD = 16
NGD = RPT // GD
NP = 10016
DSP = 12000

NB = 5
_Z = np.int32(0)
BN = N // NB

_mesh = plsc.VectorSubcoreMesh(core_axis_name="c", subcore_axis_name="s")

f32 = jnp.float32
i32 = jnp.int32


def c32(v):
    return np.int32(v)


def _iota16():
    return lax.iota(i32, 16)


def _dedup_scatter_body(keys_hbm, ids_hbm, table_hbm, kbuf, ibuf, sem):
    c = lax.axis_index("c")
    s = lax.axis_index("s")
    wid = s * c32(NC) + c
    row0w = pl.multiple_of(wid * c32(RPW), 8)

    pltpu.sync_copy(keys_hbm.at[pl.ds(row0w, RPW)], kbuf)

    def group(g, lr):
        pltpu.sync_copy(ids_hbm.at[pl.ds(row0w + lr, GA)], ibuf)
        cps = [
            pltpu.async_copy(ibuf.at[c32(j)],
                             table_hbm.at[kbuf.at[lr + c32(j)]], sem)
            for j in range(GA)
        ]
        for cp in cps:
            cp.wait()
        return lr + c32(GA)

    lax.fori_loop(np.int32(0), np.int32(NGA), group, np.int32(0))


def _dedup_scatter(keys2d, ids3):
    return pl.kernel(
        _dedup_scatter_body,
        out_type=jax.ShapeDtypeStruct((TBL + 16, 16), i32),
        mesh=_mesh,
        compiler_params=pltpu.CompilerParams(use_tc_tiling_on_sc=False),
        scratch_types=[
            pltpu.VMEM((RPW, WE), i32),
            pltpu.VMEM((GA, WE, 16), i32),
            pltpu.SemaphoreType.DMA,
        ],
    )(keys2d, ids3)


def _dedup_gather_body(keys_hbm, table_hbm, dst_hbm, dst_eff_hbm,
                       deg0_hbm, deg1_hbm,
                       kbuf, dbuf, gbuf, obuf, wbuf, zbuf, deg_sp,
                       sem, semd):
    c = lax.axis_index("c")
    s = lax.axis_index("s")
    wid = s * c32(NC) + c

    @pl.when(s == c32(0))
    def _zero():
        def zrow(k, off):
            zbuf[pl.ds(off, 16)] = jnp.zeros((16,), f32)
            return off + c32(16)
        lax.fori_loop(np.int32(0), np.int32(125), zrow, np.int32(0))
        for k in range(DSP // 2000):
            pltpu.sync_copy(zbuf, deg_sp.at[pl.ds(k * 2000, 2000)])

    plsc.subcore_barrier()

    def group(g, row0):
        row0 = pl.multiple_of(row0, 8)
        pltpu.sync_copy(keys_hbm.at[pl.ds(row0, GA)], kbuf)
        pltpu.sync_copy(dst_hbm.at[pl.ds(row0, GA)], dbuf)
        cps = [
            pltpu.async_copy(table_hbm.at[kbuf.at[c32(j)]], gbuf.at[c32(j)],
                             sem)
            for j in range(GA)
        ]
        for cp in cps:
            cp.wait()
        for j in range(GA):
            rg = row0 + c32(j)
            for q in range(WE // 16):
                sl = pl.ds(q * 16, 16)
                rep = gbuf[j, sl]
                exp = rg * c32(WE) + c32(q * 16) + _iota16()
                keep = rep == exp
                dump = c32(N) + _iota16()
                obuf[j, sl] = jnp.where(keep, dbuf[j, sl], dump)
                wbuf[j, sl] = jnp.where(keep, f32(1.0), f32(0.0))
        pltpu.sync_copy(obuf, dst_eff_hbm.at[pl.ds(row0, GA)])
        dps = [
            pltpu.async_copy(wbuf.at[c32(j)], deg_sp.at[dbuf.at[c32(j)]],
                             semd, add=True)
            for j in range(GA)
        ]
        for cp in dps:
            cp.wait()
        return row0 + c32(GA)

    lax.fori_loop(np.int32(0), np.int32(NGA), group, wid * c32(RPW))

    plsc.subcore_barrier()

    @pl.when(s == c32(0))
    def _readout():
        for k in range(N // 2000):
            pltpu.sync_copy(deg_sp.at[pl.ds(k * 2000, 2000)], zbuf)

            @pl.when(c == c32(0))
            def _w0():
                pltpu.sync_copy(zbuf, deg0_hbm.at[pl.ds(k * 2000, 2000)])

            @pl.when(c == c32(1))
            def _w1():
                pltpu.sync_copy(zbuf, deg1_hbm.at[pl.ds(k * 2000, 2000)])


def _dedup_gather(keys2d, table, dst2d):
    return pl.kernel(
        _dedup_gather_body,
        out_type=[
            jax.ShapeDtypeStruct((ROWSP, WE), i32),
            jax.ShapeDtypeStruct((N,), f32),
            jax.ShapeDtypeStruct((N,), f32),
        ],
        mesh=_mesh,
        scratch_types=[
            pltpu.VMEM((GA, WE), i32),
            pltpu.VMEM((GA, WE), i32),
            pltpu.VMEM((GA, WE), i32),
            pltpu.VMEM((GA, WE), i32),
            pltpu.VMEM((GA, WE), f32),
            pltpu.VMEM((2000,), f32),
            pltpu.VMEM_SHARED((DSP,), f32),
            pltpu.SemaphoreType.DMA,
            pltpu.SemaphoreType.DMA,
        ],
    )(keys2d, table, dst2d)


def _scale_body(dega_ref, degb_ref, x_ref, dinv_ref, y0_ref, y1_ref):
    deg = dega_ref[0, 0, :] + degb_ref[0, 0, :] + f32(1.0)
    dinv = lax.rsqrt(deg)
    dinv_ref[0, 0, :] = dinv
    y = dinv[:, None] * x_ref[...]
    y0_ref[...] = y[:, :HC]
    y1_ref[...] = y[:, HC:]


def _scale_stage(deg0, deg1, x):
    dega3 = deg0.reshape(NB, 1, BN)
    degb3 = deg1.reshape(NB, 1, BN)
    return pl.pallas_call(
        _scale_body,
        grid=(NB,),
        in_specs=[
            pl.BlockSpec((1, 1, BN), lambda i: (i, _Z, _Z)),
            pl.BlockSpec((1, 1, BN), lambda i: (i, _Z, _Z)),
            pl.BlockSpec((BN, CH), lambda i: (i, _Z)),
        ],
        out_specs=[
            pl.BlockSpec((1, 1, BN), lambda i: (i, _Z, _Z)),
            pl.BlockSpec((BN, HC), lambda i: (i, _Z)),
            pl.BlockSpec((BN, HC), lambda i: (i, _Z)),
        ],
        out_shape=[
            jax.ShapeDtypeStruct((NB, 1, BN), f32),
            jax.ShapeDtypeStruct((N, HC), f32),
            jax.ShapeDtypeStruct((N, HC), f32),
        ],
    )(dega3, degb3, x)


def _agg_half(src_hbm, dst_hbm, y_hbm, out2d, acc, sbuf, dbuf, rbuf,
              sem_i, sem_g, sem_s, s):
    def zrow(k, kk):
        for q in range(HC // 16):
            rbuf[c32(0), kk, pl.ds(q * 16, 16)] = jnp.zeros((16,), f32)
        return kk + c32(1)
    lax.fori_loop(np.int32(0), np.int32(128), zrow, np.int32(0))

    def zblk_full(bi, blk):
        @pl.when(blk < c32(NP // 128))
        def _():
            start = pl.multiple_of(blk * c32(128), 128)
            pltpu.sync_copy(rbuf.at[c32(0)], acc.at[pl.ds(start, 128)])
        return blk + c32(NS)
    lax.fori_loop(np.int32(0), np.int32((NP // 128 + NS - 1) // NS),
                  zblk_full, s)

    @pl.when(s == c32(NS - 1))
    def _ztail():
        t0 = (NP // 128) * 128
        pltpu.sync_copy(rbuf.at[c32(0), pl.ds(0, NP - t0)],
                        acc.at[pl.ds(t0, NP - t0)])

    plsc.subcore_barrier()

    def group(g, row0):
        row0 = pl.multiple_of(row0, 8)
        cs = pltpu.async_copy(src_hbm.at[pl.ds(row0, GD)], sbuf, sem_i)
        cd = pltpu.async_copy(dst_hbm.at[pl.ds(row0, GD)], dbuf, sem_i)
        cs.wait()
        cd.wait()
        gets = [pltpu.async_copy(y_hbm.at[sbuf.at[c32(0)]],
                                 rbuf.at[c32(0)], sem_g)]
        puts = []
        for j in range(GD):
            gets[j].wait()
            puts.append(pltpu.async_copy(rbuf.at[c32(j % 2)],
                                         acc.at[dbuf.at[c32(j)]], sem_s,
                                         add=True))
            if j + 1 < GD:
                if j >= 1:
                    puts[j - 1].wait()
                gets.append(pltpu.async_copy(y_hbm.at[sbuf.at[c32(j + 1)]],
                                             rbuf.at[c32((j + 1) % 2)],
                                             sem_g))
        puts[GD - 2].wait()
        puts[GD - 1].wait()
        return row0 + c32(GD)

    lax.fori_loop(np.int32(0), np.int32(NGD), group, s * c32(RPT))

    plsc.subcore_barrier()

    def wblk(bi, blk):
        @pl.when(blk < c32(N // 128))
        def _():
            start = pl.multiple_of(blk * c32(128), 128)
            pltpu.sync_copy(acc.at[pl.ds(start, 128)], rbuf.at[c32(0)])
            pltpu.sync_copy(rbuf.at[c32(0)], out2d.at[pl.ds(start, 128)])
        return blk + c32(NS)
    lax.fori_loop(np.int32(0), np.int32((N // 128 + NS - 1) // NS),
                  wblk, s)

    @pl.when(s == c32(NS - 1))
    def _wtail():
        t0 = (N // 128) * 128
        sz = N - t0
        pltpu.sync_copy(acc.at[pl.ds(t0, sz)], rbuf.at[c32(0), pl.ds(0, sz)])
        pltpu.sync_copy(rbuf.at[c32(0), pl.ds(0, sz)],
                        out2d.at[pl.ds(t0, sz)])


def _agg_body(src_hbm, dst_hbm, y0_hbm, y1_hbm, agg0_hbm, agg1_hbm,
              acc, sbuf, dbuf, rbuf, sem_i, sem_g, sem_s):
    c = lax.axis_index("c")
    s = lax.axis_index("s")

    @pl.when(c == c32(0))
    def _half0():
        _agg_half(src_hbm, dst_hbm, y0_hbm, agg0_hbm, acc, sbuf, dbuf,
                  rbuf, sem_i, sem_g, sem_s, s)

    @pl.when(c == c32(1))
    def _half1():
        _agg_half(src_hbm, dst_hbm, y1_hbm, agg1_hbm, acc, sbuf, dbuf,
                  rbuf, sem_i, sem_g, sem_s, s)


def _agg_stage(src2d, dst_eff2d, y0, y1):
    return pl.kernel(
        _agg_body,
        out_type=[
            jax.ShapeDtypeStruct((N, HC), f32),
            jax.ShapeDtypeStruct((N, HC), f32),
        ],
        mesh=_mesh,
        scratch_types=[
            pltpu.VMEM_SHARED((NP, HC), f32),
            pltpu.VMEM((GD, WE), i32),
            pltpu.VMEM((GD, WE), i32),
            pltpu.VMEM((2, WE, HC), f32),
            pltpu.SemaphoreType.DMA,
            pltpu.SemaphoreType.DMA,
            pltpu.SemaphoreType.DMA,
        ],
    )(src2d, dst_eff2d, y0, y1)


def _dense_body(x_ref, agg0_ref, agg1_ref, dinv_ref, batch_ref, w1_ref,
                out_ref, acc_ref, cnt_ref):
    i = pl.program_id(0)

    @pl.when(i == 0)
    def _init():
        acc_ref[...] = jnp.zeros_like(acc_ref)
        cnt_ref[...] = jnp.zeros_like(cnt_ref)

    dinv = dinv_ref[0, 0, :]
    agg = jnp.concatenate([agg0_ref[...], agg1_ref[...]], axis=1)
    xb = x_ref[...]
    d = dinv[:, None]
    a = d * agg + (d * d) * xb
    h = (1.0 - ALPHA) * a + ALPHA * xb
    r = lax.broadcasted_iota(i32, (CH, CH), 0)
    cc = lax.broadcasted_iota(i32, (CH, CH), 1)
    w1p = (f32(BETA) * w1_ref[...]
           + jnp.where(r == cc, f32(1.0 - BETA), f32(0.0)))
    t = jnp.dot(h, w1p, preferred_element_type=f32)
    o = jnp.maximum(t, 0.0)

    b = batch_ref[0, 0, :]
    onehot = (b[:, None] == lax.broadcasted_iota(i32, (BN, G), 1))
    onehot = onehot.astype(f32)
    acc_ref[...] += lax.dot_general(
        onehot, o, (((0,), (0,)), ((), ())), preferred_element_type=f32)
    cnt_ref[...] += jnp.sum(onehot, axis=0, keepdims=True)

    @pl.when(i == NB - 1)
    def _fin():
        cnts = jnp.maximum(cnt_ref[...], 1.0)
        out_ref[...] = acc_ref[...] / cnts[0, :][:, None]


def _dense_stage(x, agg0, agg1, dinv3, batch_i32, W1):
    batch3 = batch_i32.reshape(NB, 1, BN)
    return pl.pallas_call(
        _dense_body,
        grid=(NB,),
        in_specs=[
            pl.BlockSpec((BN, CH), lambda i: (i, _Z)),
            pl.BlockSpec((BN, HC), lambda i: (i, _Z)),
            pl.BlockSpec((BN, HC), lambda i: (i, _Z)),
            pl.BlockSpec((1, 1, BN), lambda i: (i, _Z, _Z)),
            pl.BlockSpec((1, 1, BN), lambda i: (i, _Z, _Z)),
            pl.BlockSpec((CH, CH), lambda i: (_Z, _Z)),
        ],
        out_specs=pl.BlockSpec((G, CH), lambda i: (_Z, _Z)),
        out_shape=jax.ShapeDtypeStruct((G, CH), f32),
        scratch_shapes=[
            pltpu.VMEM((G, CH), f32),
            pltpu.VMEM((1, G), f32),
        ],
    )(x, agg0, agg1, dinv3, batch3, W1)


def kernel(x, edge_index, batch, W1):
    x = x.astype(f32)
    ei = edge_index.astype(i32)
    src2 = jnp.concatenate([ei[0], ei[1]])
    dst2 = jnp.concatenate([ei[1], ei[0]])
    batch_i32 = batch.astype(i32)

    pad = E2P - E2
    pidx = jnp.arange(pad, dtype=i32)
    src2p = jnp.concatenate([src2, pidx % N])
    dst2p = jnp.concatenate([dst2, N + (pidx % 16)])
    keysp = jnp.concatenate([src2 * N + dst2, TBL + (pidx % 16)])

    keys2d = keysp.reshape(ROWSP, WE)
    keys16_2d = (keysp * 16).reshape(ROWSP, WE)
    src2d = src2p.reshape(ROWSP, WE)
    dst2d = dst2p.reshape(ROWSP, WE)
    ids3 = jnp.broadcast_to(
        jnp.arange(E2P, dtype=i32).reshape(ROWSP, WE)[:, :, None],
        (ROWSP, WE, 16))

    table = _dedup_scatter(keys2d, ids3)
    dst_eff2d, deg0, deg1 = _dedup_gather(keys16_2d, table.reshape(-1),
                                          dst2d)
    dinv3, y0, y1 = _scale_stage(deg0, deg1, x)
    agg0, agg1 = _agg_stage(src2d, dst_eff2d, y0, y1)
    return _dense_stage(x, agg0, agg1, dinv3, batch_i32, W1)
